# baseline (speedup 1.0000x reference)
import functools
import math

import jax
import jax.numpy as jnp
from jax import lax
from jax.experimental import pallas as pl
from jax.experimental.pallas import tpu as pltpu
from jax.experimental.pallas import tpu_sc as plsc

D_MODEL_K = 4096
MAX_NORM_K = math.sqrt(D_MODEL_K)
CHUNK = 1024
NBUF = 3
_LANES = 16


def _rsqrt_scalar(s):
    i = lax.bitcast_convert_type(s, jnp.int32)
    i = jnp.int32(0x5F3759DF) - lax.shift_right_arithmetic(i, 1)
    y = lax.bitcast_convert_type(i, jnp.float32)
    for _ in range(4):
        y = y * (1.5 - 0.5 * s * y * y)
    return y


def _emb_sc_kernel(ts_hbm, tbl_hbm, emb_hbm, idx_v, rows_v, acc_v, sem, *,
                   n_batch, d_model):
    nc = plsc.get_sparse_core_info().num_cores
    wid = lax.axis_index("s") * nc + lax.axis_index("c")
    nvec = d_model // _LANES

    @pl.when(wid == 0)
    def _():
        pltpu.sync_copy(ts_hbm, idx_v)
        pltpu.async_copy(tbl_hbm.at[idx_v], rows_v, sem).wait()
        for b in range(n_batch):
            def sumsq(j, acc):
                v = rows_v[b, pl.ds(j * _LANES, _LANES)]
                return acc + v * v
            acc = lax.fori_loop(0, nvec, sumsq,
                                jnp.zeros((_LANES,), jnp.float32))
            s = acc[0]
            for k in range(1, _LANES):
                s = s + acc[k]
            scale = jnp.where(s > jnp.float32(MAX_NORM_K * MAX_NORM_K),
                              MAX_NORM_K * _rsqrt_scalar(s),
                              jnp.float32(1.0))

            def rescale(j, c):
                sl = pl.ds(j * _LANES, _LANES)
                rows_v[b, sl] = rows_v[b, sl] * scale
                return c
            lax.fori_loop(0, nvec, rescale, 0)
        pltpu.sync_copy(rows_v, emb_hbm)


def _embed_max_norm(timesteps, table):
    B = timesteps.shape[0]
    D = table.shape[1]
    mesh = plsc.VectorSubcoreMesh(core_axis_name="c", subcore_axis_name="s")
    return pl.kernel(
        functools.partial(_emb_sc_kernel, n_batch=B, d_model=D),
        out_type=jax.ShapeDtypeStruct((B, D), table.dtype),
        mesh=mesh,
        scratch_types=[
            pltpu.VMEM((B,), jnp.int32),
            pltpu.VMEM((B, D), table.dtype),
            pltpu.VMEM((_LANES,), jnp.float32),
            pltpu.SemaphoreType.DMA,
        ],
    )(timesteps, table)


def _pipeline_kernel(x_hbm, emb_hbm, o_hbm, buf, emb_ref, in_sems, out_sems,
                     emb_sem, *, n_chunks, chunks_per_b):
    emb_cp = pltpu.make_async_copy(emb_hbm, emb_ref, emb_sem)
    emb_cp.start()

    def copy_in(c, slot):
        return pltpu.make_async_copy(
            x_hbm.at[pl.ds(c * CHUNK, CHUNK), :],
            buf.at[slot],
            in_sems.at[slot],
        )

    def copy_out(c, slot):
        return pltpu.make_async_copy(
            buf.at[slot],
            o_hbm.at[pl.ds(c * CHUNK, CHUNK), :],
            out_sems.at[slot],
        )

    for s in range(min(NBUF, n_chunks)):
        copy_in(s, s).start()
    emb_cp.wait()

    for c in range(n_chunks):
        slot = c % NBUF
        b = c // chunks_per_b
        copy_in(c, slot).wait()
        buf[slot] += emb_ref[pl.ds(b, 1), :]
        copy_out(c, slot).start()
        nxt = c + NBUF
        if nxt < n_chunks:
            copy_out(c, slot).wait()
            copy_in(nxt, slot).start()

    for c in range(max(0, n_chunks - NBUF), n_chunks):
        copy_out(c, c % NBUF).wait()


def kernel(x, timesteps, table):
    B, S, D = x.shape
    x2 = x.reshape(B * S, D)
    emb = _embed_max_norm(timesteps, table)
    n_chunks = (B * S) // CHUNK
    chunks_per_b = S // CHUNK
    body = functools.partial(_pipeline_kernel, n_chunks=n_chunks,
                             chunks_per_b=chunks_per_b)
    out = pl.pallas_call(
        body,
        grid=(1,),
        in_specs=[
            pl.BlockSpec(memory_space=pl.ANY),
            pl.BlockSpec(memory_space=pl.ANY),
        ],
        out_specs=pl.BlockSpec(memory_space=pl.ANY),
        scratch_shapes=[
            pltpu.VMEM((NBUF, CHUNK, D), x.dtype),
            pltpu.VMEM((B, D), x.dtype),
            pltpu.SemaphoreType.DMA((NBUF,)),
            pltpu.SemaphoreType.DMA((NBUF,)),
            pltpu.SemaphoreType.DMA,
        ],
        out_shape=jax.ShapeDtypeStruct(x2.shape, x.dtype),
    )(x2, emb)
    return out.reshape(B, S, D)
